# Initial kernel scaffold; baseline (speedup 1.0000x reference)
import jax
import jax.numpy as jnp
from jax.experimental import pallas as pl


def kernel(mbd1_loc_data, mbd1_conf_data, mbd2_loc_data, mbd2_conf_data, priors, targets):
    raise NotImplementedError("write your pallas kernel here")


# speedup vs baseline: 18.5220x; 18.5220x over previous
import jax
import jax.numpy as jnp
from jax import lax
from jax.experimental import pallas as pl
from jax.experimental.pallas import tpu as pltpu

_NCLS = 21
_B, _P, _NO = 32, 8732, 8
_TH = 0.5
_V0, _V1 = 0.1, 0.2
_NEGPOS = 3
_MAXBITS = 0x7F7FFFFF


def _mb_kernel(tgt_ref, pri_ref, loc_ref, conf_ref, out_ref,
               rk_ref, ll_ref, pr_ref, np_ref, kk_ref):
    b = pl.program_id(0)

    cx = pri_ref[0:1, :]
    cy = pri_ref[1:2, :]
    pw = pri_ref[2:3, :]
    ph = pri_ref[3:4, :]
    px1 = cx - pw * 0.5
    py1 = cy - ph * 0.5
    px2 = cx + pw * 0.5
    py2 = cy + ph * 0.5
    area_p = (px2 - px1) * (py2 - py1)

    lane = lax.broadcasted_iota(jnp.int32, (1, _P), 1)

    tx1 = [tgt_ref[0, t, 0] for t in range(_NO)]
    ty1 = [tgt_ref[0, t, 1] for t in range(_NO)]
    tx2 = [tgt_ref[0, t, 2] for t in range(_NO)]
    ty2 = [tgt_ref[0, t, 3] for t in range(_NO)]
    tlb = [tgt_ref[0, t, 4] for t in range(_NO)]

    bto = jnp.full((1, _P), -1.0, dtype=jnp.float32)
    bti = jnp.zeros((1, _P), dtype=jnp.int32)
    bpi = []
    for t in range(_NO):
        ix1 = jnp.maximum(tx1[t], px1)
        iy1 = jnp.maximum(ty1[t], py1)
        ix2 = jnp.minimum(tx2[t], px2)
        iy2 = jnp.minimum(ty2[t], py2)
        iw = jnp.maximum(ix2 - ix1, 0.0)
        ih = jnp.maximum(iy2 - iy1, 0.0)
        inter = iw * ih
        area_t = (tx2[t] - tx1[t]) * (ty2[t] - ty1[t])
        ov = inter / (area_t + area_p - inter)
        upd = ov > bto
        bti = jnp.where(upd, t, bti)
        bto = jnp.where(upd, ov, bto)
        m = jnp.max(ov)
        bpi.append(jnp.min(jnp.where(ov == m, lane, _P)))

    for t in range(_NO):
        msk = lane == bpi[t]
        bto = jnp.where(msk, 2.0, bto)
        bti = jnp.where(msk, t, bti)

    mx1 = jnp.zeros((1, _P), jnp.float32)
    my1 = jnp.zeros((1, _P), jnp.float32)
    mx2 = jnp.zeros((1, _P), jnp.float32)
    my2 = jnp.zeros((1, _P), jnp.float32)
    mlb = jnp.zeros((1, _P), jnp.float32)
    for t in range(_NO):
        sel = bti == t
        mx1 = jnp.where(sel, tx1[t], mx1)
        my1 = jnp.where(sel, ty1[t], my1)
        mx2 = jnp.where(sel, tx2[t], mx2)
        my2 = jnp.where(sel, ty2[t], my2)
        mlb = jnp.where(sel, tlb[t], mlb)

    conf_t = (mlb + 1.0).astype(jnp.int32)
    conf_t = jnp.where(bto < _TH, 0, conf_t)
    pos = conf_t > 0
    posf = pos.astype(jnp.float32)

    g0 = ((mx1 + mx2) * 0.5 - cx) / (_V0 * pw)
    g1 = ((my1 + my2) * 0.5 - cy) / (_V0 * ph)
    g2 = jnp.log((mx2 - mx1) / pw) / _V1
    g3 = jnp.log((my2 - my1) / ph) / _V1
    ll = jnp.float32(0.0)
    for i, g in enumerate((g0, g1, g2, g3)):
        d = loc_ref[0, i:i + 1, :] - g
        ad = jnp.abs(d)
        sl1 = jnp.where(ad < 1.0, 0.5 * d * d, ad - 0.5)
        ll = ll + jnp.sum(sl1 * posf)

    x = conf_ref[0]
    mxv = jnp.max(x, axis=0, keepdims=True)
    s = jnp.sum(jnp.exp(x - mxv), axis=0, keepdims=True)
    lse = jnp.log(s) + mxv
    cls = lax.broadcasted_iota(jnp.int32, (_NCLS, _P), 0)
    gathered = jnp.sum(jnp.where(cls == conf_t, x, 0.0), axis=0, keepdims=True)
    r = lse - gathered
    rk = jnp.where(pos, 0.0, r)

    num_pos = jnp.sum(conf_t > 0, dtype=jnp.int32)
    k = jnp.minimum(_NEGPOS * num_pos, _P - 1)
    pos_r = jnp.sum(r * posf)

    rk_ref[pl.ds(b, 1), :] = rk
    ll_ref[pl.ds(b, 1), :] = jnp.full((1, 1), ll, jnp.float32)
    pr_ref[pl.ds(b, 1), :] = jnp.full((1, 1), pos_r, jnp.float32)
    np_ref[pl.ds(b, 1), :] = jnp.full((1, 1), num_pos, jnp.int32)
    kk_ref[pl.ds(b, 1), :] = jnp.full((1, 1), k, jnp.int32)

    @pl.when(b == _B - 1)
    def _():
        rk_all = rk_ref[...]
        bits = lax.bitcast_convert_type(rk_all, jnp.int32)
        kv = kk_ref[...]

        def it(_, lohi):
            lo, hi = lohi
            mid = lo + lax.shift_right_logical(hi - lo + 1, 1)
            cnt = jnp.sum((bits >= mid).astype(jnp.int32), axis=1, keepdims=True)
            ok = cnt >= kv
            return jnp.where(ok, mid, lo), jnp.where(ok, hi, mid - 1)

        lo0 = jnp.zeros((_B, 1), jnp.int32)
        hi0 = jnp.full((_B, 1), _MAXBITS, jnp.int32)
        lo, _hi = lax.fori_loop(0, 31, it, (lo0, hi0))

        tf = lax.bitcast_convert_type(lo, jnp.float32)
        gt = bits > lo
        cnt_gt = jnp.sum(gt.astype(jnp.float32), axis=1, keepdims=True)
        s_gt = jnp.sum(jnp.where(gt, rk_all, 0.0), axis=1, keepdims=True)
        kf = kv.astype(jnp.float32)
        extra = s_gt + (kf - cnt_gt) * tf
        extra = jnp.where(kv > 0, extra, 0.0)

        n_total = jnp.sum(np_ref[...].astype(jnp.float32))
        ll_total = jnp.sum(ll_ref[...])
        lc_total = jnp.sum(pr_ref[...]) + jnp.sum(extra)
        out_ref[0, 0] = ll_total / n_total
        out_ref[0, 1] = lc_total / n_total


def _run(loc, conf, pri, targets, interpret=False):
    return pl.pallas_call(
        _mb_kernel,
        grid=(_B,),
        in_specs=[
            pl.BlockSpec((1, _NO, 5), lambda b: (b, 0, 0),
                         memory_space=pltpu.SMEM),
            pl.BlockSpec((4, _P), lambda b: (0, 0)),
            pl.BlockSpec((1, 4, _P), lambda b: (b, 0, 0)),
            pl.BlockSpec((1, _NCLS, _P), lambda b: (b, 0, 0)),
        ],
        out_specs=pl.BlockSpec((1, 2), lambda b: (0, 0),
                               memory_space=pltpu.SMEM),
        out_shape=jax.ShapeDtypeStruct((1, 2), jnp.float32),
        scratch_shapes=[
            pltpu.VMEM((_B, _P), jnp.float32),
            pltpu.VMEM((_B, 1), jnp.float32),
            pltpu.VMEM((_B, 1), jnp.float32),
            pltpu.VMEM((_B, 1), jnp.int32),
            pltpu.VMEM((_B, 1), jnp.int32),
        ],
        interpret=interpret,
    )(targets, pri, loc, conf)


def kernel(mbd1_loc_data, mbd1_conf_data, mbd2_loc_data, mbd2_conf_data,
           priors, targets):
    del mbd2_loc_data, mbd2_conf_data
    loc_t = jnp.transpose(mbd1_loc_data, (0, 2, 1))
    conf_t = jnp.transpose(mbd1_conf_data, (0, 2, 1))
    pri_t = priors.T
    out = _run(loc_t, conf_t, pri_t, targets)
    return out[0, 0], out[0, 1]


# speedup vs baseline: 23.1062x; 1.2475x over previous
import jax
import jax.numpy as jnp
from jax import lax
from jax.experimental import pallas as pl
from jax.experimental.pallas import tpu as pltpu

_NCLS = 21
_B, _P, _NO = 32, 8732, 8
_PPAD = 9216
_S, _L = 8, 1152
_TH = 0.5
_V0, _V1 = 0.1, 0.2
_NEGPOS = 3
_MAXBITS = 0x7F7FFFFF


def _mb_kernel(tgt_ref, pri_ref, loc_ref, conf_ref, out_ref,
               rk_ref, ll_ref, pr_ref, np_ref, kk_ref):
    b = pl.program_id(0)

    cx = pri_ref[0]
    cy = pri_ref[1]
    pw = pri_ref[2]
    ph = pri_ref[3]
    px1 = cx - pw * 0.5
    py1 = cy - ph * 0.5
    px2 = cx + pw * 0.5
    py2 = cy + ph * 0.5
    area_p = (px2 - px1) * (py2 - py1)

    pg = (lax.broadcasted_iota(jnp.int32, (_S, _L), 0) * _L
          + lax.broadcasted_iota(jnp.int32, (_S, _L), 1))

    tx1 = [tgt_ref[0, t, 0] for t in range(_NO)]
    ty1 = [tgt_ref[0, t, 1] for t in range(_NO)]
    tx2 = [tgt_ref[0, t, 2] for t in range(_NO)]
    ty2 = [tgt_ref[0, t, 3] for t in range(_NO)]
    tlb = [tgt_ref[0, t, 4] for t in range(_NO)]

    bto = jnp.full((_S, _L), -1.0, dtype=jnp.float32)
    bti = jnp.zeros((_S, _L), dtype=jnp.int32)
    bpi = []
    for t in range(_NO):
        ix1 = jnp.maximum(tx1[t], px1)
        iy1 = jnp.maximum(ty1[t], py1)
        ix2 = jnp.minimum(tx2[t], px2)
        iy2 = jnp.minimum(ty2[t], py2)
        iw = jnp.maximum(ix2 - ix1, 0.0)
        ih = jnp.maximum(iy2 - iy1, 0.0)
        inter = iw * ih
        area_t = (tx2[t] - tx1[t]) * (ty2[t] - ty1[t])
        ov = inter / (area_t + area_p - inter)
        upd = ov > bto
        bti = jnp.where(upd, t, bti)
        bto = jnp.where(upd, ov, bto)
        m = jnp.max(ov)
        bpi.append(jnp.min(jnp.where(ov == m, pg, _PPAD)))

    for t in range(_NO):
        msk = pg == bpi[t]
        bto = jnp.where(msk, 2.0, bto)
        bti = jnp.where(msk, t, bti)

    mx1 = jnp.zeros((_S, _L), jnp.float32)
    my1 = jnp.zeros((_S, _L), jnp.float32)
    mx2 = jnp.zeros((_S, _L), jnp.float32)
    my2 = jnp.zeros((_S, _L), jnp.float32)
    mlb = jnp.zeros((_S, _L), jnp.float32)
    for t in range(_NO):
        sel = bti == t
        mx1 = jnp.where(sel, tx1[t], mx1)
        my1 = jnp.where(sel, ty1[t], my1)
        mx2 = jnp.where(sel, tx2[t], mx2)
        my2 = jnp.where(sel, ty2[t], my2)
        mlb = jnp.where(sel, tlb[t], mlb)

    conf_t = (mlb + 1.0).astype(jnp.int32)
    conf_t = jnp.where(bto < _TH, 0, conf_t)
    pos = conf_t > 0
    posf = pos.astype(jnp.float32)

    g0 = ((mx1 + mx2) * 0.5 - cx) / (_V0 * pw)
    g1 = ((my1 + my2) * 0.5 - cy) / (_V0 * ph)
    g2 = jnp.log((mx2 - mx1) / pw) / _V1
    g3 = jnp.log((my2 - my1) / ph) / _V1
    ll = jnp.float32(0.0)
    for i, g in enumerate((g0, g1, g2, g3)):
        d = loc_ref[0, i] - g
        ad = jnp.abs(d)
        sl1 = jnp.where(ad < 1.0, 0.5 * d * d, ad - 0.5)
        ll = ll + jnp.sum(sl1 * posf)

    x = conf_ref[0]
    mxv = jnp.max(x, axis=0)
    s = jnp.sum(jnp.exp(x - mxv[None]), axis=0)
    lse = jnp.log(s) + mxv
    cls = lax.broadcasted_iota(jnp.int32, (_NCLS, _S, _L), 0)
    gathered = jnp.sum(jnp.where(cls == conf_t[None], x, 0.0), axis=0)
    r = lse - gathered
    valid = pg < _P
    rk = jnp.where(pos | ~valid, 0.0, r)

    num_pos = jnp.sum(conf_t > 0, dtype=jnp.int32)
    k = jnp.minimum(_NEGPOS * num_pos, _P - 1)
    pos_r = jnp.sum(r * posf)

    rk_ref[pl.ds(b, 1)] = rk[None]
    ll_ref[pl.ds(b, 1), :] = jnp.full((1, 1), ll, jnp.float32)
    pr_ref[pl.ds(b, 1), :] = jnp.full((1, 1), pos_r, jnp.float32)
    np_ref[pl.ds(b, 1), :] = jnp.full((1, 1), num_pos, jnp.int32)
    kk_ref[pl.ds(b, 1), :] = jnp.full((1, 1), k, jnp.int32)

    @pl.when(b == _B - 1)
    def _():
        rk_all = rk_ref[...]
        bits = lax.bitcast_convert_type(rk_all, jnp.int32)
        kv = kk_ref[...][:, :, None]

        def it(_, lohi):
            lo, hi = lohi
            mid = lo + lax.shift_right_logical(hi - lo + 1, 1)
            cnt = jnp.sum((bits >= mid).astype(jnp.int32), axis=2,
                          keepdims=True)
            cnt = jnp.sum(cnt, axis=1, keepdims=True)
            ok = cnt >= kv
            return jnp.where(ok, mid, lo), jnp.where(ok, hi, mid - 1)

        lo0 = jnp.zeros((_B, 1, 1), jnp.int32)
        hi0 = jnp.full((_B, 1, 1), _MAXBITS, jnp.int32)
        lo, _hi = lax.fori_loop(0, 31, it, (lo0, hi0))

        tf = lax.bitcast_convert_type(lo, jnp.float32)
        gt = bits > lo
        cnt_gt = jnp.sum(gt.astype(jnp.float32), axis=(1, 2), keepdims=True)
        s_gt = jnp.sum(jnp.where(gt, rk_all, 0.0), axis=(1, 2), keepdims=True)
        kf = kv.astype(jnp.float32)
        extra = s_gt + (kf - cnt_gt) * tf
        extra = jnp.where(kv > 0, extra, 0.0)

        n_total = jnp.sum(np_ref[...].astype(jnp.float32))
        ll_total = jnp.sum(ll_ref[...])
        lc_total = jnp.sum(pr_ref[...]) + jnp.sum(extra)
        out_ref[0, 0] = ll_total / n_total
        out_ref[0, 1] = lc_total / n_total


def _run(loc, conf, pri, targets, interpret=False):
    return pl.pallas_call(
        _mb_kernel,
        grid=(_B,),
        in_specs=[
            pl.BlockSpec((1, _NO, 5), lambda b: (b, 0, 0),
                         memory_space=pltpu.SMEM),
            pl.BlockSpec((4, _S, _L), lambda b: (0, 0, 0)),
            pl.BlockSpec((1, 4, _S, _L), lambda b: (b, 0, 0, 0)),
            pl.BlockSpec((1, _NCLS, _S, _L), lambda b: (b, 0, 0, 0)),
        ],
        out_specs=pl.BlockSpec((1, 2), lambda b: (0, 0),
                               memory_space=pltpu.SMEM),
        out_shape=jax.ShapeDtypeStruct((1, 2), jnp.float32),
        scratch_shapes=[
            pltpu.VMEM((_B, _S, _L), jnp.float32),
            pltpu.VMEM((_B, 1), jnp.float32),
            pltpu.VMEM((_B, 1), jnp.float32),
            pltpu.VMEM((_B, 1), jnp.int32),
            pltpu.VMEM((_B, 1), jnp.int32),
        ],
        interpret=interpret,
    )(targets, pri, loc, conf)


def _prep(mbd1_loc_data, mbd1_conf_data, priors):
    npad = _PPAD - _P
    pad_rows = jnp.tile(
        jnp.array([[-100.0, -100.0, 1.0, 1.0]], jnp.float32), (npad, 1))
    pri = jnp.concatenate([priors, pad_rows], axis=0)
    pri = pri.T.reshape(4, _S, _L)
    loc = jnp.pad(mbd1_loc_data, ((0, 0), (0, npad), (0, 0)))
    loc = jnp.transpose(loc, (0, 2, 1)).reshape(_B, 4, _S, _L)
    conf = jnp.pad(mbd1_conf_data, ((0, 0), (0, npad), (0, 0)))
    conf = jnp.transpose(conf, (0, 2, 1)).reshape(_B, _NCLS, _S, _L)
    return loc, conf, pri


def kernel(mbd1_loc_data, mbd1_conf_data, mbd2_loc_data, mbd2_conf_data,
           priors, targets):
    del mbd2_loc_data, mbd2_conf_data
    loc, conf, pri = _prep(mbd1_loc_data, mbd1_conf_data, priors)
    out = _run(loc, conf, pri, targets)
    return out[0, 0], out[0, 1]
